# Initial kernel scaffold; baseline (speedup 1.0000x reference)
import functools

import jax
import jax.numpy as jnp
from jax import lax
from jax.experimental import pallas as pl
from jax.experimental.pallas import tpu as pltpu
from jax.experimental.pallas import tpu_sc as plsc

_PAD = 1
_ROW_LEN = 8192
_D = 1024
_CHUNK_ELEMS = 1024
_K = 32
_NCH = _CHUNK_ELEMS // _K


def _sc_body(inp_hbm, table_hbm, out_hbm, row_v, pos_v, buf0, buf1,
             sg0, sg1):
    c = lax.axis_index("c")
    s = lax.axis_index("s")
    chunk = c * 16 + s
    row = chunk // (_ROW_LEN // _CHUNK_ELEMS)
    p = chunk % (_ROW_LEN // _CHUNK_ELEMS)

    row_base = pl.multiple_of(row * _ROW_LEN, _ROW_LEN)
    pltpu.sync_copy(inp_hbm.at[pl.ds(row_base, _ROW_LEN)], row_v)

    def acc_body(i, acc):
        x = row_v[pl.ds(i * 16, 16)]
        return acc + (x != _PAD).astype(jnp.int32)

    acc = lax.fori_loop(0, p * (_CHUNK_ELEMS // 16), acc_body,
                        jnp.zeros((16,), jnp.int32))
    offset = jnp.sum(acc)

    chunk_base = p * _CHUNK_ELEMS

    def pos_body(i, carry):
        x = row_v[pl.ds(chunk_base + i * 16, 16)]
        m = (x != _PAD).astype(jnp.int32)
        cs = plsc.cumsum(m) + carry
        pos_v[pl.ds(i * 16, 16)] = cs * m
        return carry + jnp.sum(m)

    lax.fori_loop(0, _CHUNK_ELEMS // 16, pos_body, offset)

    out_base = chunk * _CHUNK_ELEMS
    bufs = (buf0, buf1)
    sgs = (sg0, sg1)

    def start_gather(g, b):
        idx = pos_v.at[pl.ds(pl.multiple_of(g * _K, _K), _K)]
        pltpu.async_copy(table_hbm.at[idx], bufs[b], sgs[b])

    def wait_gather(b):
        pltpu.make_async_copy(table_hbm.at[pl.ds(0, _K)], bufs[b],
                              sgs[b]).wait()

    def write_out(g, b):
        dst = out_hbm.at[pl.ds(pl.multiple_of(out_base + g * _K, _K), _K)]
        pltpu.sync_copy(bufs[b], dst)

    start_gather(0, 0)
    start_gather(1, 1)

    def pipe_body(i, carry):
        for b in range(2):
            g = 2 * i + b
            wait_gather(b)
            write_out(g, b)
            start_gather(g + 2, b)
        return carry

    lax.fori_loop(0, _NCH // 2 - 1, pipe_body, 0)
    for b in range(2):
        wait_gather(b)
        write_out(_NCH - 2 + b, b)


@jax.jit
def _lpe(flat_inp, table):
    n_tokens = flat_inp.shape[0]
    mesh = plsc.VectorSubcoreMesh(core_axis_name="c", subcore_axis_name="s")
    call = functools.partial(
        pl.kernel,
        mesh=mesh,
        out_type=jax.ShapeDtypeStruct((n_tokens, _D), jnp.float32),
        scratch_types=[
            pltpu.VMEM((_ROW_LEN,), jnp.int32),
            pltpu.VMEM((_CHUNK_ELEMS,), jnp.int32),
            pltpu.VMEM((_K, _D), jnp.float32),
            pltpu.VMEM((_K, _D), jnp.float32),
            pltpu.SemaphoreType.DMA,
            pltpu.SemaphoreType.DMA,
        ],
    )(_sc_body)
    return call(flat_inp, table)


def kernel(input, table):
    b, l = input.shape
    out = _lpe(input.reshape(-1), table)
    return out.reshape(b, l, table.shape[1])


# speedup vs baseline: 2.3038x; 2.3038x over previous
import functools

import jax
import jax.numpy as jnp
from jax import lax
from jax.experimental import pallas as pl
from jax.experimental.pallas import tpu as pltpu
from jax.experimental.pallas import tpu_sc as plsc

_PAD = 1
_ROW_LEN = 8192
_D = 1024
_CHUNK_ELEMS = 1024
_K = 32
_NCH = _CHUNK_ELEMS // _K


def _sc_body(inp_hbm, table_hbm, out_hbm, row_v, pos_v, buf0, buf1,
             sg0, sg1):
    c = lax.axis_index("c")
    s = lax.axis_index("s")
    chunk = c * 16 + s
    row = chunk // (_ROW_LEN // _CHUNK_ELEMS)
    p = chunk % (_ROW_LEN // _CHUNK_ELEMS)

    row_base = pl.multiple_of(row * _ROW_LEN, _ROW_LEN)
    pltpu.sync_copy(inp_hbm.at[pl.ds(row_base, _ROW_LEN)], row_v)

    def acc_body(i, acc):
        x = row_v[pl.ds(i * 16, 16)]
        return acc + jnp.minimum(jnp.abs(x - jnp.int32(_PAD)), jnp.int32(1))

    acc = lax.fori_loop(0, p * (_CHUNK_ELEMS // 16), acc_body,
                        jnp.zeros((16,), jnp.int32))
    offset = jnp.sum(acc)

    chunk_base = p * _CHUNK_ELEMS

    def pos_body(i, carry):
        x = row_v[pl.ds(chunk_base + i * 16, 16)]
        m = jnp.minimum(jnp.abs(x - jnp.int32(_PAD)), jnp.int32(1))
        cs = plsc.cumsum(m) + carry
        pos_v[pl.ds(i * 16, 16)] = cs * m
        return carry + jnp.sum(m)

    lax.fori_loop(0, _CHUNK_ELEMS // 16, pos_body, offset)

    out_base = chunk * _CHUNK_ELEMS
    bufs = (buf0, buf1)
    sgs = (sg0, sg1)

    def start_gather(g, b):
        idx = pos_v.at[pl.ds(pl.multiple_of(g * _K, _K), _K)]
        pltpu.async_copy(table_hbm.at[idx], bufs[b], sgs[b])

    def wait_gather(b):
        pltpu.make_async_copy(table_hbm.at[pl.ds(0, _K)], bufs[b],
                              sgs[b]).wait()

    def write_out(g, b):
        dst = out_hbm.at[pl.ds(pl.multiple_of(out_base + g * _K, _K), _K)]
        pltpu.sync_copy(bufs[b], dst)

    start_gather(0, 0)
    start_gather(1, 1)

    def pipe_body(i, carry):
        for b in range(2):
            g = 2 * i + b
            wait_gather(b)
            write_out(g, b)
            start_gather(g + 2, b)
        return carry

    lax.fori_loop(0, _NCH // 2 - 1, pipe_body, 0)
    for b in range(2):
        wait_gather(b)
        write_out(_NCH - 2 + b, b)


@jax.jit
def _lpe(flat_inp, table):
    n_tokens = flat_inp.shape[0]
    mesh = plsc.VectorSubcoreMesh(core_axis_name="c", subcore_axis_name="s")
    call = functools.partial(
        pl.kernel,
        mesh=mesh,
        out_type=jax.ShapeDtypeStruct((n_tokens, _D), jnp.float32),
        scratch_types=[
            pltpu.VMEM((_ROW_LEN,), jnp.int32),
            pltpu.VMEM((_CHUNK_ELEMS,), jnp.int32),
            pltpu.VMEM((_K, _D), jnp.float32),
            pltpu.VMEM((_K, _D), jnp.float32),
            pltpu.SemaphoreType.DMA,
            pltpu.SemaphoreType.DMA,
        ],
        compiler_params=pltpu.CompilerParams(needs_layout_passes=False),
    )(_sc_body)
    return call(flat_inp, table)


def kernel(input, table):
    b, l = input.shape
    out = _lpe(input.reshape(-1), table)
    return out.reshape(b, l, table.shape[1])


# speedup vs baseline: 2.3141x; 1.0045x over previous
import functools

import jax
import jax.numpy as jnp
from jax import lax
from jax.experimental import pallas as pl
from jax.experimental.pallas import tpu as pltpu
from jax.experimental.pallas import tpu_sc as plsc

_PAD = 1
_ROW_LEN = 8192
_D = 1024
_CHUNK_ELEMS = 1024
_K = 32
_NCH = _CHUNK_ELEMS // _K
_NBUF = 3


def _sc_body(inp_hbm, table_hbm, out_hbm, row_v, pos_v, buf0, buf1, buf2,
             sg0, sg1, sg2, so0, so1, so2):
    c = lax.axis_index("c")
    s = lax.axis_index("s")
    chunk = c * 16 + s
    row = chunk // (_ROW_LEN // _CHUNK_ELEMS)
    p = chunk % (_ROW_LEN // _CHUNK_ELEMS)

    row_base = pl.multiple_of(row * _ROW_LEN, _ROW_LEN)
    pltpu.sync_copy(inp_hbm.at[pl.ds(row_base, _ROW_LEN)], row_v)

    def acc_body(i, acc):
        x = row_v[pl.ds(i * 16, 16)]
        return acc + jnp.minimum(jnp.abs(x - jnp.int32(_PAD)), jnp.int32(1))

    acc = lax.fori_loop(0, p * (_CHUNK_ELEMS // 16), acc_body,
                        jnp.zeros((16,), jnp.int32))
    offset = jnp.sum(acc)

    chunk_base = p * _CHUNK_ELEMS

    def pos_body(i, carry):
        x = row_v[pl.ds(chunk_base + i * 16, 16)]
        m = jnp.minimum(jnp.abs(x - jnp.int32(_PAD)), jnp.int32(1))
        cs = plsc.cumsum(m) + carry
        pos_v[pl.ds(i * 16, 16)] = cs * m
        return carry + jnp.sum(m)

    lax.fori_loop(0, _CHUNK_ELEMS // 16, pos_body, offset)

    out_base = chunk * _CHUNK_ELEMS
    bufs = (buf0, buf1, buf2)
    sgs = (sg0, sg1, sg2)
    sos = (so0, so1, so2)

    def start_gather(g, b):
        idx = pos_v.at[pl.ds(pl.multiple_of(g * _K, _K), _K)]
        pltpu.async_copy(table_hbm.at[idx], bufs[b], sgs[b])

    def wait_gather(b):
        pltpu.make_async_copy(table_hbm.at[pl.ds(0, _K)], bufs[b],
                              sgs[b]).wait()

    def start_write(g, b):
        dst = out_hbm.at[pl.ds(pl.multiple_of(out_base + g * _K, _K), _K)]
        pltpu.async_copy(bufs[b], dst, sos[b])

    def wait_write(b):
        pltpu.make_async_copy(bufs[b], out_hbm.at[pl.ds(0, _K)],
                              sos[b]).wait()

    start_gather(0, 0)
    start_gather(1, 1)
    wait_gather(0)
    start_write(0, 0)
    start_gather(2, 2)

    def pipe_body(i, carry):
        for b in range(_NBUF):
            g = _NBUF * i + 1 + b
            wait_gather((1 + b) % _NBUF)
            start_write(g, (1 + b) % _NBUF)
            wait_write(b)

            @pl.when(g + 2 < _NCH)
            def _():
                start_gather(g + 2, b)
        return carry

    lax.fori_loop(0, (_NCH - 2) // _NBUF, pipe_body, 0)

    g_last = _NCH - 1
    wait_gather(g_last % _NBUF)
    start_write(g_last, g_last % _NBUF)
    wait_write((g_last - 1) % _NBUF)
    wait_write(g_last % _NBUF)


@jax.jit
def _lpe(flat_inp, table):
    n_tokens = flat_inp.shape[0]
    mesh = plsc.VectorSubcoreMesh(core_axis_name="c", subcore_axis_name="s")
    call = functools.partial(
        pl.kernel,
        mesh=mesh,
        out_type=jax.ShapeDtypeStruct((n_tokens, _D), jnp.float32),
        scratch_types=[
            pltpu.VMEM((_ROW_LEN,), jnp.int32),
            pltpu.VMEM((_CHUNK_ELEMS,), jnp.int32),
            pltpu.VMEM((_K, _D), jnp.float32),
            pltpu.VMEM((_K, _D), jnp.float32),
            pltpu.VMEM((_K, _D), jnp.float32),
            pltpu.SemaphoreType.DMA,
            pltpu.SemaphoreType.DMA,
            pltpu.SemaphoreType.DMA,
            pltpu.SemaphoreType.DMA,
            pltpu.SemaphoreType.DMA,
            pltpu.SemaphoreType.DMA,
        ],
        compiler_params=pltpu.CompilerParams(needs_layout_passes=False),
    )(_sc_body)
    return call(flat_inp, table)


def kernel(input, table):
    b, l = input.shape
    out = _lpe(input.reshape(-1), table)
    return out.reshape(b, l, table.shape[1])


# speedup vs baseline: 2.3237x; 1.0042x over previous
import functools

import jax
import jax.numpy as jnp
from jax import lax
from jax.experimental import pallas as pl
from jax.experimental.pallas import tpu as pltpu
from jax.experimental.pallas import tpu_sc as plsc

_PAD = 1
_ROW_LEN = 8192
_D = 1024
_CHUNK_ELEMS = 1024
_K = 32
_NCH = _CHUNK_ELEMS // _K
_NBUF = 3


def _sc_body(inp_hbm, table_hbm, out_hbm, row_v, pos_v, buf0, buf1, buf2,
             carry_s, sg0, sg1, sg2, so0, so1, so2):
    c = lax.axis_index("c")
    s = lax.axis_index("s")
    chunk = c * 16 + s
    row = chunk // (_ROW_LEN // _CHUNK_ELEMS)
    p = chunk % (_ROW_LEN // _CHUNK_ELEMS)

    row_base = pl.multiple_of(row * _ROW_LEN, _ROW_LEN)
    pltpu.sync_copy(inp_hbm.at[pl.ds(row_base, _ROW_LEN)], row_v)

    def mask16(off):
        x = row_v[pl.ds(off, 16)]
        return jnp.minimum(jnp.abs(x - jnp.int32(_PAD)), jnp.int32(1))

    def acc_body(i, acc):
        for j in range(4):
            acc = acc + mask16(i * 64 + j * 16)
        return acc

    acc = lax.fori_loop(0, p * (_CHUNK_ELEMS // 64), acc_body,
                        jnp.zeros((16,), jnp.int32))
    carry_s[0] = jnp.sum(acc)

    chunk_base = p * _CHUNK_ELEMS

    def compute_piece(q):
        carry = carry_s[0]
        for j in range(2):
            m = mask16(chunk_base + q * _K + j * 16)
            cs = plsc.cumsum(m) + carry
            pos_v[pl.ds(q * _K + j * 16, 16)] = cs * m
            carry = carry + jnp.sum(m)
        carry_s[0] = carry

    out_base = chunk * _CHUNK_ELEMS
    bufs = (buf0, buf1, buf2)
    sgs = (sg0, sg1, sg2)
    sos = (so0, so1, so2)

    def start_gather(g, b):
        idx = pos_v.at[pl.ds(pl.multiple_of(g * _K, _K), _K)]
        pltpu.async_copy(table_hbm.at[idx], bufs[b], sgs[b])

    def wait_gather(b):
        pltpu.make_async_copy(table_hbm.at[pl.ds(0, _K)], bufs[b],
                              sgs[b]).wait()

    def start_write(g, b):
        dst = out_hbm.at[pl.ds(pl.multiple_of(out_base + g * _K, _K), _K)]
        pltpu.async_copy(bufs[b], dst, sos[b])

    def wait_write(b):
        pltpu.make_async_copy(bufs[b], out_hbm.at[pl.ds(0, _K)],
                              sos[b]).wait()

    compute_piece(0)
    start_gather(0, 0)
    compute_piece(1)
    start_gather(1, 1)
    compute_piece(2)
    wait_gather(0)
    start_write(0, 0)
    start_gather(2, 2)

    def pipe_body(i, carry):
        for b in range(_NBUF):
            g = _NBUF * i + 1 + b

            @pl.when(g + 2 < _NCH)
            def _():
                compute_piece(g + 2)

            wait_gather((1 + b) % _NBUF)
            start_write(g, (1 + b) % _NBUF)
            wait_write(b)

            @pl.when(g + 2 < _NCH)
            def _():
                start_gather(g + 2, b)
        return carry

    lax.fori_loop(0, (_NCH - 2) // _NBUF, pipe_body, 0)

    g_last = _NCH - 1
    wait_gather(g_last % _NBUF)
    start_write(g_last, g_last % _NBUF)
    wait_write((g_last - 1) % _NBUF)
    wait_write(g_last % _NBUF)


@jax.jit
def _lpe(flat_inp, table):
    n_tokens = flat_inp.shape[0]
    mesh = plsc.VectorSubcoreMesh(core_axis_name="c", subcore_axis_name="s")
    call = functools.partial(
        pl.kernel,
        mesh=mesh,
        out_type=jax.ShapeDtypeStruct((n_tokens, _D), jnp.float32),
        scratch_types=[
            pltpu.VMEM((_ROW_LEN,), jnp.int32),
            pltpu.VMEM((_CHUNK_ELEMS,), jnp.int32),
            pltpu.VMEM((_K, _D), jnp.float32),
            pltpu.VMEM((_K, _D), jnp.float32),
            pltpu.VMEM((_K, _D), jnp.float32),
            pltpu.SMEM((1,), jnp.int32),
            pltpu.SemaphoreType.DMA,
            pltpu.SemaphoreType.DMA,
            pltpu.SemaphoreType.DMA,
            pltpu.SemaphoreType.DMA,
            pltpu.SemaphoreType.DMA,
            pltpu.SemaphoreType.DMA,
        ],
        compiler_params=pltpu.CompilerParams(needs_layout_passes=False),
    )(_sc_body)
    return call(flat_inp, table)


def kernel(input, table):
    b, l = input.shape
    out = _lpe(input.reshape(-1), table)
    return out.reshape(b, l, table.shape[1])
